# baseline (speedup 1.0000x reference)
import functools

import jax
import jax.numpy as jnp
from jax import lax
from jax.experimental import pallas as pl
from jax.experimental.pallas import tpu as pltpu
from jax.experimental.pallas import tpu_sc as plsc

_S = 64.0
_MS = 0.35 * 64.0

_ROWS = 1024
_COLS = 100000
_G = _COLS // 8
_L = 16
_NBUF = 3
_NLAB = _ROWS // _L


def _make_sc_kernel():
    info = plsc.get_sparse_core_info()
    nw = info.num_cores * info.num_subcores
    max_cnt = _G // nw + 1
    flag_words = ((max_cnt + _L - 1) // _L + 1) * _L
    mesh = plsc.VectorSubcoreMesh(core_axis_name="c", subcore_axis_name="s")

    @functools.partial(
        pl.kernel,
        out_type=jax.ShapeDtypeStruct((_COLS, _ROWS), jnp.float32),
        mesh=mesh,
        compiler_params=pltpu.CompilerParams(needs_layout_passes=False),
        scratch_types=(
            [pltpu.VMEM((_ROWS,), jnp.int32),
             pltpu.VMEM((flag_words,), jnp.int32)]
            + [pltpu.VMEM((8, _ROWS), jnp.float32) for _ in range(2 * _NBUF)]
            + [pltpu.SemaphoreType.DMA for _ in range(2 * _NBUF)]
        ),
    )
    def sc_kernel(x_hbm, lab_hbm, out_hbm,
                  lab_v, flag_v, ib0, ib1, ib2, ob0, ob1, ob2,
                  is0, is1, is2, os0, os1, os2):
        ibufs, obufs = [ib0, ib1, ib2], [ob0, ob1, ob2]
        isems, osems = [is0, is1, is2], [os0, os1, os2]
        cid = lax.axis_index("c")
        sid = lax.axis_index("s")
        wid = sid * info.num_cores + cid
        g0 = wid * _G // nw
        g1 = (wid + 1) * _G // nw
        cnt = g1 - g0
        pltpu.sync_copy(lab_hbm, lab_v)

        def zero(i, c):
            flag_v[pl.ds(i * _L, _L)] = jnp.zeros((_L,), jnp.int32)
            return c

        lax.fori_loop(0, flag_words // _L, zero, 0)
        ones = jnp.ones((_L,), jnp.int32)

        def mark(j, c):
            labs = lab_v[pl.ds(j * _L, _L)]
            q = lax.div(labs, 8) - g0
            m = (labs >= g0 * 8) & (labs < g1 * 8)
            plsc.store_scatter(flag_v, [jnp.where(m, q, 0)], ones, mask=m)
            return c

        lax.fori_loop(0, _NLAB, mark, 0)

        lanes = lax.iota(jnp.int32, _L)

        def in_copy(q, b):
            return pltpu.make_async_copy(
                x_hbm.at[pl.ds((g0 + q) * 8, 8)], ibufs[b], isems[b])

        def out_copy(q, b):
            return pltpu.make_async_copy(
                obufs[b], out_hbm.at[pl.ds((g0 + q) * 8, 8)], osems[b])

        def compute(q, b):
            ib, ob = ibufs[b], obufs[b]

            def body(i, c):
                for rr in range(8):
                    ob[rr, pl.ds(i * _L, _L)] = ib[rr, pl.ds(i * _L, _L)] * _S
                return c

            lax.fori_loop(0, _ROWS // _L, body, 0)

            fbase = (q // _L) * _L
            fl = flag_v[pl.ds(fbase, _L)]
            flag = jnp.max(jnp.where(lanes == q - fbase, fl, 0))

            @pl.when(flag > 0)
            def _():
                c0 = (g0 + q) * 8

                def fix(j, c):
                    labs = lab_v[pl.ds(j * _L, _L)]
                    m = (labs >= c0) & (labs < c0 + 8)
                    hit = jnp.max(jnp.where(m, 1, 0))

                    @pl.when(hit > 0)
                    def _():
                        for rr in range(8):
                            m_rr = m & (labs == c0 + rr)
                            v = ob[rr, pl.ds(j * _L, _L)]
                            ob[rr, pl.ds(j * _L, _L)] = \
                                jnp.where(m_rr, v - _MS, v)

                    return c

                lax.fori_loop(0, _NLAB, fix, 0)

        for b in range(_NBUF):
            @pl.when(b < cnt)
            def _(b=b):
                in_copy(b, b).start()

        def outer(tt, carry):
            for b in range(_NBUF):
                q = tt * _NBUF + b

                @pl.when(q < cnt)
                def _():
                    @pl.when(q >= _NBUF)
                    def _():
                        out_copy(q - _NBUF, b).wait()

                    in_copy(q, b).wait()
                    compute(q, b)
                    out_copy(q, b).start()

                    @pl.when(q + _NBUF < cnt)
                    def _():
                        in_copy(q + _NBUF, b).start()

            return carry

        lax.fori_loop(0, (max_cnt + _NBUF - 1) // _NBUF, outer, 0)

        for b in range(_NBUF):
            @pl.when(jnp.maximum(cnt - _NBUF, 0) + b < cnt)
            def _(b=b):
                out_copy(0, b).wait()

    return sc_kernel


_sc_kernel = _make_sc_kernel()


@jax.jit
def kernel(logits, labels):
    out_t = _sc_kernel(logits.T, labels.astype(jnp.int32))
    return out_t.T


# speedup vs baseline: 1.0563x; 1.0563x over previous
import functools

import jax
import jax.numpy as jnp
from jax import lax
from jax.experimental import pallas as pl
from jax.experimental.pallas import tpu as pltpu
from jax.experimental.pallas import tpu_sc as plsc

_S = 64.0
_MS = 0.35 * 64.0

_ROWS = 1024
_COLS = 100000
_CR = 16
_G = _COLS // _CR
_L = 16
_NBUF = 3
_NLAB = _ROWS // _L


def _make_sc_kernel():
    info = plsc.get_sparse_core_info()
    nw = info.num_cores * info.num_subcores
    max_cnt = _G // nw + 1
    flag_words = ((max_cnt + _L - 1) // _L + 1) * _L
    mesh = plsc.VectorSubcoreMesh(core_axis_name="c", subcore_axis_name="s")

    @functools.partial(
        pl.kernel,
        out_type=jax.ShapeDtypeStruct((_COLS, _ROWS), jnp.float32),
        mesh=mesh,
        compiler_params=pltpu.CompilerParams(needs_layout_passes=False),
        scratch_types=(
            [pltpu.VMEM((_ROWS,), jnp.int32),
             pltpu.VMEM((flag_words,), jnp.int32)]
            + [pltpu.VMEM((_CR, _ROWS), jnp.float32) for _ in range(2 * _NBUF)]
            + [pltpu.SemaphoreType.DMA for _ in range(2 * _NBUF)]
        ),
    )
    def sc_kernel(x_hbm, lab_hbm, out_hbm,
                  lab_v, flag_v, ib0, ib1, ib2, ob0, ob1, ob2,
                  is0, is1, is2, os0, os1, os2):
        ibufs, obufs = [ib0, ib1, ib2], [ob0, ob1, ob2]
        isems, osems = [is0, is1, is2], [os0, os1, os2]
        cid = lax.axis_index("c")
        sid = lax.axis_index("s")
        wid = sid * info.num_cores + cid
        g0 = wid * _G // nw
        g1 = (wid + 1) * _G // nw
        cnt = g1 - g0
        pltpu.sync_copy(lab_hbm, lab_v)

        def zero(i, c):
            flag_v[pl.ds(i * _L, _L)] = jnp.zeros((_L,), jnp.int32)
            return c

        lax.fori_loop(0, flag_words // _L, zero, 0)
        ones = jnp.ones((_L,), jnp.int32)

        def mark(j, c):
            labs = lab_v[pl.ds(j * _L, _L)]
            q = lax.div(labs, _CR) - g0
            m = (labs >= g0 * _CR) & (labs < g1 * _CR)
            plsc.store_scatter(flag_v, [jnp.where(m, q, 0)], ones, mask=m)
            return c

        lax.fori_loop(0, _NLAB, mark, 0)

        lanes = lax.iota(jnp.int32, _L)

        def in_copy(q, b):
            return pltpu.make_async_copy(
                x_hbm.at[pl.ds((g0 + q) * _CR, _CR)], ibufs[b], isems[b])

        def out_copy(q, b):
            return pltpu.make_async_copy(
                obufs[b], out_hbm.at[pl.ds((g0 + q) * _CR, _CR)], osems[b])

        def compute(q, b):
            ib, ob = ibufs[b], obufs[b]

            def body(i, c):
                for rr in range(_CR):
                    ob[rr, pl.ds(i * _L, _L)] = ib[rr, pl.ds(i * _L, _L)] * _S
                return c

            lax.fori_loop(0, _ROWS // _L, body, 0)

            fbase = (q // _L) * _L
            fl = flag_v[pl.ds(fbase, _L)]
            flag = jnp.max(jnp.where(lanes == q - fbase, fl, 0))

            @pl.when(flag > 0)
            def _():
                c0 = (g0 + q) * _CR

                def fix(j, c):
                    labs = lab_v[pl.ds(j * _L, _L)]
                    m = (labs >= c0) & (labs < c0 + _CR)
                    hit = jnp.max(jnp.where(m, 1, 0))

                    @pl.when(hit > 0)
                    def _():
                        for rr in range(_CR):
                            m_rr = m & (labs == c0 + rr)
                            v = ob[rr, pl.ds(j * _L, _L)]
                            ob[rr, pl.ds(j * _L, _L)] = \
                                jnp.where(m_rr, v - _MS, v)

                    return c

                lax.fori_loop(0, _NLAB, fix, 0)

        for b in range(_NBUF):
            @pl.when(b < cnt)
            def _(b=b):
                in_copy(b, b).start()

        def outer(tt, carry):
            for b in range(_NBUF):
                q = tt * _NBUF + b

                @pl.when(q < cnt)
                def _():
                    @pl.when(q >= _NBUF)
                    def _():
                        out_copy(q - _NBUF, b).wait()

                    in_copy(q, b).wait()
                    compute(q, b)
                    out_copy(q, b).start()

                    @pl.when(q + _NBUF < cnt)
                    def _():
                        in_copy(q + _NBUF, b).start()

            return carry

        lax.fori_loop(0, (max_cnt + _NBUF - 1) // _NBUF, outer, 0)

        for b in range(_NBUF):
            @pl.when(jnp.maximum(cnt - _NBUF, 0) + b < cnt)
            def _(b=b):
                out_copy(0, b).wait()

    return sc_kernel


_sc_kernel = _make_sc_kernel()


@jax.jit
def kernel(logits, labels):
    out_t = _sc_kernel(logits.T, labels.astype(jnp.int32))
    return out_t.T
